# baseline (speedup 1.0000x reference)
import functools

import jax
import jax.numpy as jnp
from jax import lax
from jax.experimental import pallas as pl
from jax.experimental.pallas import tpu as pltpu
from jax.experimental.pallas import tpu_sc as plsc

LANES = 16
NC = 2
NS = 16
NW = NC * NS
CHUNK = 128
NBUF = 2


@functools.lru_cache(maxsize=None)
def _build_sc_kernel(BF: int, D: int, per_w: int):
    n_chunks = per_w // CHUNK
    assert n_chunks % NBUF == 0
    W = 128
    mesh = plsc.VectorSubcoreMesh(core_axis_name="c", subcore_axis_name="s")

    @functools.partial(
        pl.kernel,
        mesh=mesh,
        out_type=jax.ShapeDtypeStruct((BF * D,), jnp.float32),
        scratch_types=(
            [
                pltpu.VMEM((per_w,), jnp.int32),
                pltpu.VMEM((per_w,), jnp.int32),
                pltpu.VMEM((per_w,), jnp.float32),
                pltpu.VMEM((NBUF, CHUNK, W), jnp.float32),
                pltpu.VMEM((NBUF, CHUNK, W), jnp.float32),
                pltpu.VMEM((NBUF, CHUNK * D), jnp.float32),
            ]
            + [pltpu.SemaphoreType.DMA] * (2 * NBUF)
        ),
    )
    def sc_kernel(tokq_hbm, sub_hbm, inp_hbm, emb_hbm, bias_hbm, out_hbm,
                  idx_v, sub_v, inp_v, emb_v, bias_v, out_v, *sems):
        g_sem = sems[:NBUF]
        o_sem = sems[NBUF:]
        wid = lax.axis_index("s") * NC + lax.axis_index("c")
        base = wid * per_w

        pltpu.sync_copy(tokq_hbm.at[pl.ds(base, per_w)], idx_v)
        pltpu.sync_copy(sub_hbm.at[pl.ds(base, per_w)], sub_v)
        pltpu.sync_copy(inp_hbm.at[pl.ds(base, per_w)], inp_v)

        def fire_gathers(c, b):
            idx_slice = idx_v.at[pl.ds(c * CHUNK, CHUNK)]
            pltpu.async_copy(emb_hbm.at[idx_slice], emb_v.at[b], g_sem[b])
            pltpu.async_copy(bias_hbm.at[idx_slice], bias_v.at[b], g_sem[b])

        def wait_gathers(c, b):
            idx_slice = idx_v.at[pl.ds(c * CHUNK, CHUNK)]
            pltpu.make_async_copy(emb_hbm.at[idx_slice], emb_v.at[b],
                                  g_sem[b]).wait()
            pltpu.make_async_copy(bias_hbm.at[idx_slice], bias_v.at[b],
                                  g_sem[b]).wait()

        def out_copy(c, b):
            return pltpu.make_async_copy(
                out_v.at[b],
                out_hbm.at[pl.ds((base + c * CHUNK) * D, CHUNK * D)],
                o_sem[b])

        for b in range(NBUF):
            fire_gathers(b, b)

        def ring_body(g, carry):
            for b in range(NBUF):
                c = g * NBUF + b
                wait_gathers(c, b)

                @pl.when(c >= NBUF)
                def _():
                    out_copy(c - NBUF, b).wait()

                def group_body(gr, carry2):
                    row0 = gr * LANES
                    sv = inp_v[pl.ds(c * CHUNK + row0, LANES)]
                    ov = sub_v[pl.ds(c * CHUNK + row0, LANES)]
                    for r in range(LANES):
                        s = sv[r]
                        o = ov[r]
                        for h in range(D // LANES):
                            x = emb_v[b, row0 + r, pl.ds(o + h * LANES, LANES)] * s \
                                + bias_v[b, row0 + r, pl.ds(o + h * LANES, LANES)]
                            y = jnp.where(x > 0.0, x, jnp.exp(x) - 1.0)
                            out_v[b, pl.ds((row0 + r) * D + h * LANES, LANES)] = y
                    return carry2

                lax.fori_loop(0, CHUNK // LANES, group_body, 0)
                out_copy(c, b).start()

                @pl.when(c + NBUF < n_chunks)
                def _():
                    fire_gathers(c + NBUF, b)
            return carry

        lax.fori_loop(0, n_chunks // NBUF, ring_body, 0)
        for b in range(NBUF):
            out_copy(n_chunks - NBUF + b, b).wait()

    return sc_kernel


def kernel(input_tokens, inputs, embeddings, bias):
    B, F = input_tokens.shape
    V, D = embeddings.shape
    BF = B * F
    R = 128 // D
    tok = input_tokens.reshape(BF).astype(jnp.int32)
    inp = inputs.reshape(BF).astype(jnp.float32)

    quantum = NW * CHUNK * NBUF
    BFp = ((BF + quantum - 1) // quantum) * quantum
    if BFp != BF:
        tok = jnp.pad(tok, (0, BFp - BF))
        inp = jnp.pad(inp, (0, BFp - BF))

    tokq = tok // R
    sub = (tok % R) * D
    embw = embeddings.reshape(V // R, 128)
    biasw = bias.reshape(V // R, 128)

    out = _build_sc_kernel(BFp, D, BFp // NW)(tokq, sub, inp, embw, biasw)
    out = out.reshape(BFp, D)
    if BFp != BF:
        out = out[:BF]
    return out.reshape(B, F, D)


# speedup vs baseline: 1.1892x; 1.1892x over previous
import functools

import jax
import jax.numpy as jnp
from jax import lax
from jax.experimental import pallas as pl
from jax.experimental.pallas import tpu as pltpu
from jax.experimental.pallas import tpu_sc as plsc

LANES = 16
NC = 2
NS = 16
NW = NC * NS
GROUPS = 8
CHUNK = GROUPS * LANES
OBUF = 2


@functools.lru_cache(maxsize=None)
def _build_sc_kernel(BF: int, D: int, per_w: int):
    n_chunks = per_w // CHUNK
    assert n_chunks % OBUF == 0
    mesh = plsc.VectorSubcoreMesh(core_axis_name="c", subcore_axis_name="s")

    @functools.partial(
        pl.kernel,
        mesh=mesh,
        out_type=jax.ShapeDtypeStruct((BF * D,), jnp.float32),
        scratch_types=(
            [
                pltpu.VMEM((per_w,), jnp.int32),
                pltpu.VMEM((per_w,), jnp.float32),
                pltpu.VMEM((GROUPS, LANES, D), jnp.float32),
                pltpu.VMEM((GROUPS, LANES, D), jnp.float32),
                pltpu.VMEM((OBUF, CHUNK * D), jnp.float32),
            ]
            + [pltpu.SemaphoreType.DMA] * (2 * GROUPS + OBUF)
        ),
    )
    def sc_kernel(tok_hbm, inp_hbm, emb_hbm, bias_hbm, out_hbm,
                  idx_v, inp_v, emb_v, bias_v, out_v, *sems):
        e_sem = sems[:GROUPS]
        b_sem = sems[GROUPS:2 * GROUPS]
        o_sem = sems[2 * GROUPS:]
        wid = lax.axis_index("s") * NC + lax.axis_index("c")
        base = wid * per_w

        pltpu.sync_copy(tok_hbm.at[pl.ds(base, per_w)], idx_v)
        pltpu.sync_copy(inp_hbm.at[pl.ds(base, per_w)], inp_v)

        def fire_group(c, gg):
            tokv = idx_v[pl.ds(c * CHUNK + gg * LANES, LANES)]
            for r in range(LANES):
                t = tokv[r]
                pltpu.async_copy(emb_hbm.at[pl.ds(t, 1), :],
                                 emb_v.at[gg, pl.ds(r, 1), :], e_sem[gg])
                pltpu.async_copy(bias_hbm.at[pl.ds(t, 1), :],
                                 bias_v.at[gg, pl.ds(r, 1), :], b_sem[gg])

        def wait_group(gg):
            for r in range(LANES):
                pltpu.make_async_copy(emb_hbm.at[pl.ds(0, 1), :],
                                      emb_v.at[gg, pl.ds(r, 1), :],
                                      e_sem[gg]).wait()
                pltpu.make_async_copy(bias_hbm.at[pl.ds(0, 1), :],
                                      bias_v.at[gg, pl.ds(r, 1), :],
                                      b_sem[gg]).wait()

        def out_copy(c, par):
            return pltpu.make_async_copy(
                out_v.at[par],
                out_hbm.at[pl.ds((base + c * CHUNK) * D, CHUNK * D)],
                o_sem[par])

        for gg in range(GROUPS):
            fire_group(0, gg)

        def super_body(sg, carry):
            for par in range(OBUF):
                c = sg * OBUF + par

                @pl.when(c >= OBUF)
                def _():
                    out_copy(c - OBUF, par).wait()

                for gg in range(GROUPS):
                    wait_group(gg)
                    sv = inp_v[pl.ds(c * CHUNK + gg * LANES, LANES)]
                    for r in range(LANES):
                        s = sv[r]
                        for h in range(D // LANES):
                            sl = pl.ds(h * LANES, LANES)
                            x = emb_v[gg, r, sl] * s + bias_v[gg, r, sl]
                            y = jnp.where(x > 0.0, x, jnp.exp(x) - 1.0)
                            out_v[par, pl.ds((gg * LANES + r) * D + h * LANES,
                                             LANES)] = y

                    @pl.when(c + 1 < n_chunks)
                    def _():
                        fire_group(c + 1, gg)

                out_copy(c, par).start()
            return carry

        lax.fori_loop(0, n_chunks // OBUF, super_body, 0)
        for par in range(OBUF):
            out_copy(n_chunks - OBUF + par, par).wait()

    return sc_kernel


def kernel(input_tokens, inputs, embeddings, bias):
    B, F = input_tokens.shape
    V, D = embeddings.shape
    BF = B * F
    tok = input_tokens.reshape(BF).astype(jnp.int32)
    inp = inputs.reshape(BF).astype(jnp.float32)

    quantum = NW * CHUNK * OBUF
    BFp = ((BF + quantum - 1) // quantum) * quantum
    if BFp != BF:
        tok = jnp.pad(tok, (0, BFp - BF))
        inp = jnp.pad(inp, (0, BFp - BF))

    out = _build_sc_kernel(BFp, D, BFp // NW)(tok, inp, embeddings, bias)
    out = out.reshape(BFp, D)
    if BFp != BF:
        out = out[:BF]
    return out.reshape(B, F, D)
